# Initial kernel scaffold; baseline (speedup 1.0000x reference)
import jax
import jax.numpy as jnp
from jax.experimental import pallas as pl


def kernel(ids, ids_valid, ids_mask, embed_table):
    raise NotImplementedError("write your pallas kernel here")


# speedup vs baseline: 1.4953x; 1.4953x over previous
import functools

import jax
import jax.numpy as jnp
from jax import lax
from jax.experimental import pallas as pl
from jax.experimental.pallas import tpu as pltpu
from jax.experimental.pallas import tpu_sc as plsc


def _make_gather(B: int, D: int):
    info = plsc.get_sparse_core_info()
    nw = info.num_cores * info.num_subcores
    assert B % (8 * nw) == 0
    b_per_w = B // nw
    ch = 32
    n_ch = b_per_w // ch
    mesh = plsc.VectorSubcoreMesh(core_axis_name="c", subcore_axis_name="s")

    @functools.partial(
        pl.kernel,
        mesh=mesh,
        out_type=jax.ShapeDtypeStruct((B, D), jnp.float32),
        scratch_types=[
            pltpu.VMEM((b_per_w,), jnp.int32),
            pltpu.VMEM((2, ch, D), jnp.float32),
            pltpu.SemaphoreType.DMA,
            pltpu.SemaphoreType.DMA,
            pltpu.SemaphoreType.DMA,
            pltpu.SemaphoreType.DMA,
        ],
    )
    def gather(table_hbm, idx_hbm, out_hbm, idx_v, rows_v, g0, g1, w0, w1):
        gsem = (g0, g1)
        wsem = (w0, w1)
        wid = lax.axis_index("s") * info.num_cores + lax.axis_index("c")
        base = wid * b_per_w
        pltpu.sync_copy(idx_hbm.at[pl.ds(base, b_per_w)], idx_v)

        def start_gather(i):
            return pltpu.async_copy(
                table_hbm.at[idx_v.at[pl.ds(i * ch, ch)]],
                rows_v.at[i % 2], gsem[i % 2])

        def start_write(i):
            return pltpu.async_copy(
                rows_v.at[i % 2], out_hbm.at[pl.ds(base + i * ch, ch)], wsem[i % 2])

        gathers = [None] * n_ch
        writes = [None] * n_ch
        gathers[0] = start_gather(0)
        for i in range(n_ch):
            if i + 1 < n_ch:
                if i >= 1:
                    writes[i - 1].wait()
                gathers[i + 1] = start_gather(i + 1)
            gathers[i].wait()
            writes[i] = start_write(i)
        writes[n_ch - 1].wait()

    return gather


def kernel(ids, ids_valid, ids_mask, embed_table):
    vocab, d = embed_table.shape
    b, s = ids.shape
    ids_flat = jnp.clip(ids.reshape(-1), 0, vocab - 1)
    out = _make_gather(b * s, d)(embed_table, ids_flat)
    return (out.reshape(b, s, d), ids_valid, ids, ids_mask)


# speedup vs baseline: 1.5287x; 1.0224x over previous
import functools

import jax
import jax.numpy as jnp
from jax import lax
from jax.experimental import pallas as pl
from jax.experimental.pallas import tpu as pltpu
from jax.experimental.pallas import tpu_sc as plsc


def _make_gather(B: int, D: int):
    info = plsc.get_sparse_core_info()
    nw = info.num_cores * info.num_subcores
    assert B % (8 * nw) == 0
    b_per_w = B // nw
    ch = 32
    nbuf = 3
    n_ch = b_per_w // ch
    mesh = plsc.VectorSubcoreMesh(core_axis_name="c", subcore_axis_name="s")

    @functools.partial(
        pl.kernel,
        mesh=mesh,
        out_type=jax.ShapeDtypeStruct((B, D), jnp.float32),
        scratch_types=[
            pltpu.VMEM((b_per_w,), jnp.int32),
            pltpu.VMEM((nbuf, ch, D), jnp.float32),
        ] + [pltpu.SemaphoreType.DMA] * (2 * nbuf),
    )
    def gather(table_hbm, idx_hbm, out_hbm, idx_v, rows_v, *sems):
        gsem, wsem = sems[:nbuf], sems[nbuf:]
        wid = lax.axis_index("s") * info.num_cores + lax.axis_index("c")
        base = wid * b_per_w
        pltpu.sync_copy(idx_hbm.at[pl.ds(base, b_per_w)], idx_v)

        def start_gather(i):
            return pltpu.async_copy(
                table_hbm.at[idx_v.at[pl.ds(i * ch, ch)]],
                rows_v.at[i % nbuf], gsem[i % nbuf])

        def start_write(i):
            return pltpu.async_copy(
                rows_v.at[i % nbuf], out_hbm.at[pl.ds(base + i * ch, ch)],
                wsem[i % nbuf])

        gathers = [None] * n_ch
        writes = [None] * n_ch
        for j in range(min(nbuf - 1, n_ch)):
            gathers[j] = start_gather(j)
        for i in range(n_ch):
            j = i + nbuf - 1
            if j < n_ch:
                if j - nbuf >= 0:
                    writes[j - nbuf].wait()
                gathers[j] = start_gather(j)
            gathers[i].wait()
            writes[i] = start_write(i)
        for i in range(max(0, n_ch - nbuf), n_ch):
            writes[i].wait()

    return gather


def kernel(ids, ids_valid, ids_mask, embed_table):
    vocab, d = embed_table.shape
    b, s = ids.shape
    ids_flat = jnp.clip(ids.reshape(-1), 0, vocab - 1)
    out = _make_gather(b * s, d)(embed_table, ids_flat)
    return (out.reshape(b, s, d), ids_valid, ids, ids_mask)


# speedup vs baseline: 1.5371x; 1.0054x over previous
import functools

import jax
import jax.numpy as jnp
from jax import lax
from jax.experimental import pallas as pl
from jax.experimental.pallas import tpu as pltpu
from jax.experimental.pallas import tpu_sc as plsc


def _make_gather(B: int, D: int, vocab: int):
    info = plsc.get_sparse_core_info()
    nw = info.num_cores * info.num_subcores
    assert B % (8 * nw) == 0
    b_per_w = B // nw
    ch = 32
    nbuf = 3
    n_ch = b_per_w // ch
    mesh = plsc.VectorSubcoreMesh(core_axis_name="c", subcore_axis_name="s")

    @functools.partial(
        pl.kernel,
        mesh=mesh,
        out_type=jax.ShapeDtypeStruct((B, D), jnp.float32),
        scratch_types=[
            pltpu.VMEM((b_per_w,), jnp.int32),
            pltpu.VMEM((nbuf, ch, D), jnp.float32),
        ] + [pltpu.SemaphoreType.DMA] * (2 * nbuf),
    )
    def gather(table_hbm, idx_hbm, out_hbm, idx_v, rows_v, *sems):
        gsem, wsem = sems[:nbuf], sems[nbuf:]
        wid = lax.axis_index("s") * info.num_cores + lax.axis_index("c")
        base = wid * b_per_w
        pltpu.sync_copy(idx_hbm.at[pl.ds(base, b_per_w)], idx_v)
        for t in range(b_per_w // 16):
            sl = pl.ds(t * 16, 16)
            idx_v[sl] = jnp.clip(idx_v[sl], 0, vocab - 1)

        def start_gather(i):
            return pltpu.async_copy(
                table_hbm.at[idx_v.at[pl.ds(i * ch, ch)]],
                rows_v.at[i % nbuf], gsem[i % nbuf])

        def start_write(i):
            return pltpu.async_copy(
                rows_v.at[i % nbuf], out_hbm.at[pl.ds(base + i * ch, ch)],
                wsem[i % nbuf])

        gathers = [None] * n_ch
        writes = [None] * n_ch
        for j in range(min(nbuf - 1, n_ch)):
            gathers[j] = start_gather(j)
        for i in range(n_ch):
            j = i + nbuf - 1
            if j < n_ch:
                if j - nbuf >= 0:
                    writes[j - nbuf].wait()
                gathers[j] = start_gather(j)
            gathers[i].wait()
            writes[i] = start_write(i)
        for i in range(max(0, n_ch - nbuf), n_ch):
            writes[i].wait()

    return gather


def kernel(ids, ids_valid, ids_mask, embed_table):
    vocab, d = embed_table.shape
    b, s = ids.shape
    ids_flat = ids.reshape(-1)
    out = _make_gather(b * s, d, vocab)(embed_table, ids_flat)
    return (out.reshape(b, s, d), ids_valid, ids, ids_mask)


# speedup vs baseline: 1.5596x; 1.0147x over previous
import functools

import jax
import jax.numpy as jnp
from jax import lax
from jax.experimental import pallas as pl
from jax.experimental.pallas import tpu as pltpu
from jax.experimental.pallas import tpu_sc as plsc


def _make_gather(B: int, D: int, vocab: int):
    info = plsc.get_sparse_core_info()
    nw = info.num_cores * info.num_subcores
    assert B % (8 * nw) == 0
    b_per_w = B // nw
    ch = 16
    nbuf = 6
    n_ch = b_per_w // ch
    mesh = plsc.VectorSubcoreMesh(core_axis_name="c", subcore_axis_name="s")

    @functools.partial(
        pl.kernel,
        mesh=mesh,
        out_type=jax.ShapeDtypeStruct((B, D), jnp.float32),
        scratch_types=[
            pltpu.VMEM((b_per_w,), jnp.int32),
            pltpu.VMEM((nbuf, ch, D), jnp.float32),
        ] + [pltpu.SemaphoreType.DMA] * (2 * nbuf),
    )
    def gather(table_hbm, idx_hbm, out_hbm, idx_v, rows_v, *sems):
        gsem, wsem = sems[:nbuf], sems[nbuf:]
        wid = lax.axis_index("s") * info.num_cores + lax.axis_index("c")
        base = wid * b_per_w
        pltpu.sync_copy(idx_hbm.at[pl.ds(base, b_per_w)], idx_v)
        for t in range(b_per_w // 16):
            sl = pl.ds(t * 16, 16)
            idx_v[sl] = jnp.clip(idx_v[sl], 0, vocab - 1)

        def start_gather(i):
            return pltpu.async_copy(
                table_hbm.at[idx_v.at[pl.ds(i * ch, ch)]],
                rows_v.at[i % nbuf], gsem[i % nbuf])

        def start_write(i):
            return pltpu.async_copy(
                rows_v.at[i % nbuf], out_hbm.at[pl.ds(base + i * ch, ch)],
                wsem[i % nbuf])

        gathers = [None] * n_ch
        writes = [None] * n_ch
        for j in range(min(nbuf - 1, n_ch)):
            gathers[j] = start_gather(j)
        for i in range(n_ch):
            j = i + nbuf - 1
            if j < n_ch:
                if j - nbuf >= 0:
                    writes[j - nbuf].wait()
                gathers[j] = start_gather(j)
            gathers[i].wait()
            writes[i] = start_write(i)
        for i in range(max(0, n_ch - nbuf), n_ch):
            writes[i].wait()

    return gather


def kernel(ids, ids_valid, ids_mask, embed_table):
    vocab, d = embed_table.shape
    b, s = ids.shape
    ids_flat = ids.reshape(-1)
    out = _make_gather(b * s, d, vocab)(embed_table, ids_flat)
    return (out.reshape(b, s, d), ids_valid, ids, ids_mask)
